# Initial kernel scaffold; baseline (speedup 1.0000x reference)
import jax
import jax.numpy as jnp
from jax.experimental import pallas as pl


def kernel(x, edge_index, W1l, W1r, b1, W2l, W2r, b2):
    raise NotImplementedError("write your pallas kernel here")


# speedup vs baseline: 4.6771x; 4.6771x over previous
import functools

import jax
import jax.numpy as jnp
from jax import lax
from jax.experimental import pallas as pl
from jax.experimental.pallas import tpu as pltpu
from jax.experimental.pallas import tpu_sc as plsc

N = 10000
E = 160000
D = 256
DH = 128
NS = 16
NC = 2
K = 128

EPT = E // NS
NH = 2
NCHH = 40
PADF = NH * NCHH * K - EPT

EPW = E // (NC * NS)
NCHC = 40
PADC = NCHC * K - EPW

NPAD = 10112
RPT_Z = NPAD // NS
RPT = 624
REM = N - NS * RPT

_MESH = dict(core_axis_name="c", subcore_axis_name="s")


def _sc_agg(xl, xh, srcp, dstp):
  outs = (jax.ShapeDtypeStruct((N, DH), jnp.float32),
          jax.ShapeDtypeStruct((N, DH), jnp.float32))
  scratch = (
      pltpu.VMEM_SHARED((NPAD, DH), jnp.float32),
      pltpu.VMEM((NCHH, K), jnp.int32),
      pltpu.VMEM((NCHH, K), jnp.int32),
      pltpu.VMEM((2, K, DH), jnp.float32),
      pltpu.SemaphoreType.DMA,
  )
  zf = jnp.zeros((NPAD, DH), jnp.float32)

  @functools.partial(pl.kernel, out_type=outs,
                     mesh=plsc.VectorSubcoreMesh(**_MESH),
                     scratch_types=scratch)
  def k(xl_hbm, xh_hbm, src_hbm, dst_hbm, zf_hbm, out_l, out_h,
        acc, srcv, dstv, rows, gsem):
    cid = lax.axis_index("c")
    sid = lax.axis_index("s")

    z0 = sid * RPT_Z
    pltpu.sync_copy(zf_hbm.at[pl.ds(z0, RPT_Z)], acc.at[pl.ds(z0, RPT_Z)])
    plsc.subcore_barrier()

    def run(table):
      for p in range(NH):
        pltpu.sync_copy(src_hbm.at[sid, p], srcv)
        pltpu.sync_copy(dst_hbm.at[sid, p], dstv)
        pltpu.async_copy(table.at[srcv.at[0]], rows.at[0], gsem)

        @pl.loop(0, NCHH, step=2)
        def _(j):
          for b in (0, 1):
            jj = j + b

            @pl.when(jj + 1 < NCHH)
            def _():
              pltpu.async_copy(table.at[srcv.at[jj + 1]], rows.at[1 - b],
                               gsem)

            pltpu.make_async_copy(table.at[srcv.at[jj]], rows.at[b],
                                  gsem).wait()
            pltpu.sync_copy(rows.at[b], acc.at[dstv.at[jj]], add=True)

    @pl.when(cid == 0)
    def _():
      run(xl_hbm)

    @pl.when(cid == 1)
    def _():
      run(xh_hbm)

    plsc.subcore_barrier()

    o0 = sid * RPT

    def copy_out(out):
      pltpu.sync_copy(acc.at[pl.ds(o0, RPT)], out.at[pl.ds(o0, RPT)])

      @pl.when(sid == NS - 1)
      def _():
        pltpu.sync_copy(acc.at[pl.ds(NS * RPT, REM)],
                        out.at[pl.ds(NS * RPT, REM)])

    @pl.when(cid == 0)
    def _():
      copy_out(out_l)

    @pl.when(cid == 1)
    def _():
      copy_out(out_h)

  return k(xl, xh, srcp, dstp, zf)


def _sc_cnt(dstc):
  outs = (jax.ShapeDtypeStruct((N,), jnp.float32),
          jax.ShapeDtypeStruct((N,), jnp.float32))
  STG = 640
  scratch = (
      pltpu.VMEM_SHARED((NPAD,), jnp.float32),
      pltpu.VMEM((NCHC, K), jnp.int32),
      pltpu.VMEM((K,), jnp.float32),
      pltpu.VMEM((STG,), jnp.float32),
      pltpu.SemaphoreType.DMA,
  )

  @functools.partial(pl.kernel, out_type=outs,
                     mesh=plsc.VectorSubcoreMesh(**_MESH),
                     scratch_types=scratch)
  def k(dst_hbm, out0, out1, cnt, dstv, onesv, stagev, sem):
    cid = lax.axis_index("c")
    sid = lax.axis_index("s")

    @pl.loop(0, K, step=16)
    def _(i):
      onesv[pl.ds(i, 16)] = jnp.ones((16,), jnp.float32)

    @pl.loop(0, STG, step=16)
    def _(i):
      stagev[pl.ds(i, 16)] = jnp.zeros((16,), jnp.float32)

    z0 = sid * RPT_Z
    pltpu.sync_copy(stagev.at[pl.ds(0, RPT_Z)], cnt.at[pl.ds(z0, RPT_Z)])
    pltpu.sync_copy(dst_hbm.at[cid, sid], dstv)
    plsc.subcore_barrier()

    @pl.loop(0, NCHC)
    def _(j):
      pltpu.async_copy(onesv, cnt.at[dstv.at[j]], sem, add=True)

    @pl.loop(0, NCHC)
    def _(j):
      pltpu.make_async_copy(onesv, cnt.at[dstv.at[j]], sem).wait()

    plsc.subcore_barrier()

    o0 = sid * RPT

    def copy_out(out):
      pltpu.sync_copy(cnt.at[pl.ds(o0, RPT)], stagev.at[pl.ds(0, RPT)])
      pltpu.sync_copy(stagev.at[pl.ds(0, RPT)], out.at[pl.ds(o0, RPT)])

      @pl.when(sid == NS - 1)
      def _():
        pltpu.sync_copy(cnt.at[pl.ds(NS * RPT, REM)],
                        stagev.at[pl.ds(624, REM)])
        pltpu.sync_copy(stagev.at[pl.ds(624, REM)],
                        out.at[pl.ds(NS * RPT, REM)])

    @pl.when(cid == 0)
    def _():
      copy_out(out0)

    @pl.when(cid == 1)
    def _():
      copy_out(out1)

  return k(dstc)


BM = 1000


def _tc_layer(al, ah, c0, c1, xl, xh, Wl, Wr, b, final):
  def body(al_ref, ah_ref, c0_ref, c1_ref, xl_ref, xh_ref, wl_ref, wr_ref,
           b_ref, *outs):
    r = 1.0 / jnp.maximum(c0_ref[...] + c1_ref[...], 1.0)
    z = (jnp.dot(al_ref[...] * r, wl_ref[:DH, :])
         + jnp.dot(ah_ref[...] * r, wl_ref[DH:, :])
         + jnp.dot(xl_ref[...], wr_ref[:DH, :])
         + jnp.dot(xh_ref[...], wr_ref[DH:, :])
         + b_ref[...])
    h = 1.0 / (1.0 + jnp.exp(-z))
    nrm = jnp.sqrt(jnp.sum(h * h, axis=1, keepdims=True))
    h = h / jnp.maximum(nrm, 1e-12)
    if final:
      m = jnp.max(h, axis=1, keepdims=True)
      lse = jnp.log(jnp.sum(jnp.exp(h - m), axis=1, keepdims=True)) + m
      outs[0][...] = h - lse
    else:
      outs[0][...] = h[:, :DH]
      outs[1][...] = h[:, DH:]

  half = pl.BlockSpec((BM, DH), lambda i: (i, 0))
  cspec = pl.BlockSpec((BM, 1), lambda i: (i, 0))
  full = pl.BlockSpec((D, D), lambda i: (0, 0))
  in_specs = [half, half, cspec, cspec, half, half, full, full,
              pl.BlockSpec((1, D), lambda i: (0, 0))]
  if final:
    out_shape = jax.ShapeDtypeStruct((N, D), jnp.float32)
    out_specs = pl.BlockSpec((BM, D), lambda i: (i, 0))
  else:
    out_shape = (jax.ShapeDtypeStruct((N, DH), jnp.float32),
                 jax.ShapeDtypeStruct((N, DH), jnp.float32))
    out_specs = (half, half)
  return pl.pallas_call(
      body, grid=(N // BM,), in_specs=in_specs, out_specs=out_specs,
      out_shape=out_shape)(al, ah, c0, c1, xl, xh, Wl, Wr, b)


def kernel(x, edge_index, W1l, W1r, b1, W2l, W2r, b2):
  src = edge_index[0].reshape(NS, EPT)
  dst = edge_index[1].reshape(NS, EPT)
  srcp = jnp.pad(src, ((0, 0), (0, PADF))).reshape(NS, NH, NCHH, K)
  dstp = jnp.pad(dst, ((0, 0), (0, PADF)),
                 constant_values=N).reshape(NS, NH, NCHH, K)
  dstc = jnp.pad(edge_index[1].reshape(NC * NS, EPW), ((0, 0), (0, PADC)),
                 constant_values=N).reshape(NC, NS, NCHC, K)
  xl = x[:, :DH]
  xh = x[:, DH:]

  cnt0, cnt1 = _sc_cnt(dstc)
  c0 = cnt0.reshape(N, 1)
  c1 = cnt1.reshape(N, 1)
  a1l, a1h = _sc_agg(xl, xh, srcp, dstp)
  hl, hh = _tc_layer(a1l, a1h, c0, c1, xl, xh, W1l, W1r,
                     b1.reshape(1, D), False)
  a2l, a2h = _sc_agg(hl, hh, srcp, dstp)
  return _tc_layer(a2l, a2h, c0, c1, hl, hh, W2l, W2r, b2.reshape(1, D),
                   True)
